# baseline (speedup 1.0000x reference)
import functools

import jax
import jax.numpy as jnp
import numpy as np
from jax import lax
from jax.experimental import pallas as pl
from jax.experimental.pallas import tpu as pltpu
from jax.experimental.pallas import tpu_sc as plsc

N = 10000
E = 160000
D = 256
DH = 128
DE = 16

NSUB = 16
K = 80
EPT = E // NSUB
NCH = EPT // K
RPT_A = 632
RPT_LAST = N - 15 * RPT_A

EB = 2000
MB = 1000

_PERM_H = np.concatenate(
    [np.concatenate([np.arange(16) * 2 + b * 32, np.arange(16) * 2 + 1 + b * 32])
     for b in range(DH // 32)])
_PERM_FULL = np.concatenate([_PERM_H, DH + _PERM_H])


def _edge_linear(edge_attr, We, be2):
    def body(a_ref, w_ref, b_ref, o_ref):
        o_ref[...] = (jnp.dot(a_ref[...], w_ref[...],
                              preferred_element_type=jnp.float32)
                      + b_ref[0]).astype(jnp.bfloat16)

    return pl.pallas_call(
        body,
        grid=(2, E // EB),
        in_specs=[
            pl.BlockSpec((EB, DE), lambda c, i: (i, 0)),
            pl.BlockSpec((DE, DH), lambda c, i: (0, c)),
            pl.BlockSpec((1, 1, DH), lambda c, i: (c, 0, 0)),
        ],
        out_specs=pl.BlockSpec((EB, DH), lambda c, i: (c * (E // EB) + i, 0)),
        out_shape=jax.ShapeDtypeStruct((2 * E, DH), jnp.bfloat16),
    )(edge_attr, We, be2.reshape(2, 1, DH))


def _sc_aggregate(xbf, eh, srcx4, dst3):
    mesh = plsc.VectorSubcoreMesh(core_axis_name="c", subcore_axis_name="s")

    @functools.partial(
        pl.kernel,
        out_type=jax.ShapeDtypeStruct((2 * N, DH), jnp.float32),
        mesh=mesh,
        scratch_types=[
            pltpu.VMEM_SHARED((N, DH), jnp.float32),
            pltpu.VMEM((4, K), jnp.int32),
            pltpu.VMEM((4, K), jnp.int32),
            pltpu.VMEM((2, K, DH), jnp.float32),
            pltpu.VMEM((2, K, DH // 2), jnp.int32),
            pltpu.SemaphoreType.DMA,
            pltpu.SemaphoreType.DMA,
            pltpu.SemaphoreType.DMA,
            pltpu.SemaphoreType.DMA,
            pltpu.SemaphoreType.DMA,
            pltpu.SemaphoreType.DMA,
            pltpu.SemaphoreType.DMA,
            pltpu.SemaphoreType.DMA,
            pltpu.SemaphoreType.DMA,
        ],
    )
    def k(x2_hbm, eh_hbm, srcx_hbm, dst_hbm, out_hbm,
          acc, srcv, dstv, xbuf, ebuf, semx0, semx1, seme0, seme1,
          semsc, semi0, semi1, semi2, semi3):
        semx = [semx0, semx1]
        seme = [seme0, seme1]
        c = lax.axis_index("c")
        s = lax.axis_index("s")

        def zbody(r, carry):
            for g in range(DH // 16):
                xbuf[0, r, pl.ds(g * 16, 16)] = jnp.zeros((16,), jnp.float32)
            return carry
        lax.fori_loop(0, K, zbody, 0)
        row0 = s * RPT_A

        def _zero_rows(nrows):
            full = nrows // K
            for kk in range(full):
                pltpu.sync_copy(xbuf.at[0], acc.at[pl.ds(row0 + kk * K, K)])
            r = nrows - full * K
            if r:
                pltpu.sync_copy(xbuf.at[0, pl.ds(0, r)],
                                acc.at[pl.ds(row0 + full * K, r)])

        pl.when(s < NSUB - 1)(lambda: _zero_rows(RPT_A))
        pl.when(s == NSUB - 1)(lambda: _zero_rows(RPT_LAST))
        plsc.subcore_barrier()

        ebase0 = s * EPT
        semi = [semi0, semi1, semi2, semi3]
        row_hbm = c * NSUB + s

        def _idx_start(i, q):
            pltpu.async_copy(srcx_hbm.at[row_hbm, i], srcv.at[q], semi[q])
            pltpu.async_copy(dst_hbm.at[s, i], dstv.at[q], semi[q])

        def _idx_wait(q):
            pltpu.make_async_copy(srcx_hbm.at[0, 0], srcv.at[q], semi[q]).wait()
            pltpu.make_async_copy(dst_hbm.at[0, 0], dstv.at[q], semi[q]).wait()

        def _start(i, q, p):
            pltpu.async_copy(x2_hbm.at[srcv.at[q]], xbuf.at[p], semx[p])
            pltpu.async_copy(eh_hbm.at[pl.ds(c * E + ebase0 + i * K, K)],
                             ebuf.at[p], seme[p])

        _idx_start(0, 0)
        _idx_start(1, 1)
        _idx_wait(0)
        _start(0, 0, 0)

        def step(i, b, first=False):
            p = b % 2
            pn = 1 - p
            qn1 = (b + 1) % 4
            qn2 = (b + 2) % 4

            wait_sc = lambda: pltpu.make_async_copy(
                xbuf.at[pn], acc.at[dstv.at[pn]], semsc).wait()
            if first:
                pl.when(i > 0)(wait_sc)
            else:
                wait_sc()
            pl.when(i + 2 < NCH)(lambda: _idx_start(i + 2, qn2))

            def _pref():
                _idx_wait(qn1)
                _start(i + 1, qn1, pn)
            pl.when(i + 1 < NCH)(_pref)

            pltpu.make_async_copy(
                x2_hbm.at[srcv.at[b]], xbuf.at[p], semx[p]).wait()
            pltpu.make_async_copy(
                eh_hbm.at[pl.ds(0, K)], ebuf.at[p], seme[p]).wait()

            def rbody(r, cr):
                for g in range(DH // 32):
                    ev = ebuf[p, r, pl.ds(g * 16, 16)]
                    ee = lax.bitcast_convert_type(ev << 16, jnp.float32)
                    eo = lax.bitcast_convert_type(ev & (-65536), jnp.float32)
                    xe = xbuf[p, r, pl.ds(g * 32, 16)]
                    xo = xbuf[p, r, pl.ds(g * 32 + 16, 16)]
                    xbuf[p, r, pl.ds(g * 32, 16)] = jnp.maximum(xe + ee, 0.0)
                    xbuf[p, r, pl.ds(g * 32 + 16, 16)] = jnp.maximum(xo + eo, 0.0)
                return cr
            lax.fori_loop(0, K, rbody, 0)

            pltpu.async_copy(xbuf.at[p], acc.at[dstv.at[b]], semsc, add=True)

        def quad(j, carry):
            i0 = j * 4
            step(i0, 0, first=True)
            step(i0 + 1, 1)
            step(i0 + 2, 2)
            step(i0 + 3, 3)
            return carry
        lax.fori_loop(0, NCH // 4, quad, 0)
        step(NCH - 1, (NCH - 1) % 4)

        pltpu.make_async_copy(
            xbuf.at[(NCH - 1) % 2], acc.at[dstv.at[0]], semsc).wait()
        plsc.subcore_barrier()

        def _writeout(nrows):
            pltpu.sync_copy(acc.at[pl.ds(row0, nrows)],
                            out_hbm.at[pl.ds(c * N + row0, nrows)])

        pl.when(s < NSUB - 1)(lambda: _writeout(RPT_A))
        pl.when(s == NSUB - 1)(lambda: _writeout(RPT_LAST))

    return k(xbf, eh, srcx4, dst3)


def _mlp(x, h2, W1, W1P, b1r, W2, b2r):
    def body(x_ref, al_ref, ar_ref, w1_ref, w1p_ref, b1_ref, w2_ref, b2_ref,
             o_ref):
        w1p = w1p_ref[...]
        t = jnp.dot(x_ref[...], w1_ref[...], preferred_element_type=jnp.float32)
        t = t + jnp.dot(al_ref[...], w1p[:DH, :],
                        preferred_element_type=jnp.float32)
        t = t + jnp.dot(ar_ref[...], w1p[DH:, :],
                        preferred_element_type=jnp.float32)
        t = jnp.maximum(t + b1_ref[...], 0.0)
        o_ref[...] = jnp.dot(t, w2_ref[...],
                             preferred_element_type=jnp.float32) + b2_ref[...]

    return pl.pallas_call(
        body,
        grid=(N // MB,),
        in_specs=[
            pl.BlockSpec((MB, D), lambda i: (i, 0)),
            pl.BlockSpec((MB, DH), lambda i: (i, 0)),
            pl.BlockSpec((MB, DH), lambda i: (N // MB + i, 0)),
            pl.BlockSpec((D, D), lambda i: (0, 0)),
            pl.BlockSpec((D, D), lambda i: (0, 0)),
            pl.BlockSpec((1, D), lambda i: (0, 0)),
            pl.BlockSpec((D, D), lambda i: (0, 0)),
            pl.BlockSpec((1, D), lambda i: (0, 0)),
        ],
        out_specs=pl.BlockSpec((MB, D), lambda i: (i, 0)),
        out_shape=jax.ShapeDtypeStruct((N, D), jnp.float32),
    )(x, h2, h2, W1, W1P, b1r, W2, b2r)


def kernel(x, edge_index, edge_attr, We, be, W1, b1, W2, b2):
    src = edge_index[0].astype(jnp.int32)
    dst = edge_index[1].astype(jnp.int32)
    src2 = src * 2
    srcx4 = jnp.concatenate([src2, src2 + 1]).reshape(2 * NSUB, NCH, K)
    dst3 = dst.reshape(NSUB, NCH, K)
    x2p = x[:, _PERM_FULL].reshape(2 * N, DH)

    eh = _edge_linear(edge_attr, We, be.reshape(2, DH))
    eh = lax.bitcast_convert_type(
        eh.reshape(2 * E, DH // 2, 2), jnp.int32)
    h2 = _sc_aggregate(x2p, eh, srcx4, dst3)
    W1P = W1[_PERM_FULL, :]
    return _mlp(x, h2, W1, W1P, b1.reshape(1, D), W2, b2.reshape(1, D))


# speedup vs baseline: 1.0176x; 1.0176x over previous
import functools

import jax
import jax.numpy as jnp
import numpy as np
from jax import lax
from jax.experimental import pallas as pl
from jax.experimental.pallas import tpu as pltpu
from jax.experimental.pallas import tpu_sc as plsc

N = 10000
E = 160000
D = 256
DH = 128
DE = 16

NSUB = 16
K = 80
EPT = E // NSUB
NCH = EPT // K
RPT_A = 632
RPT_LAST = N - 15 * RPT_A

EB = 2000
MB = 1000

_PERM_INTL = np.concatenate(
    [b * 32 + np.stack([np.arange(16), 16 + np.arange(16)], axis=1).ravel()
     for b in range(D // 32)])


def _edge_linear(edge_attr, We, be2):
    def body(a_ref, w_ref, b_ref, o_ref):
        o_ref[...] = (jnp.dot(a_ref[...], w_ref[...],
                              preferred_element_type=jnp.float32)
                      + b_ref[0]).astype(jnp.bfloat16)

    return pl.pallas_call(
        body,
        grid=(2, E // EB),
        in_specs=[
            pl.BlockSpec((EB, DE), lambda c, i: (i, 0)),
            pl.BlockSpec((DE, DH), lambda c, i: (0, c)),
            pl.BlockSpec((1, 1, DH), lambda c, i: (c, 0, 0)),
        ],
        out_specs=pl.BlockSpec((EB, DH), lambda c, i: (c * (E // EB) + i, 0)),
        out_shape=jax.ShapeDtypeStruct((2 * E, DH), jnp.bfloat16),
    )(edge_attr, We, be2.reshape(2, 1, DH))


def _sc_aggregate(xbf, eh, srcx4, dst3):
    mesh = plsc.VectorSubcoreMesh(core_axis_name="c", subcore_axis_name="s")

    @functools.partial(
        pl.kernel,
        out_type=jax.ShapeDtypeStruct((2 * N, DH), jnp.float32),
        mesh=mesh,
        scratch_types=[
            pltpu.VMEM_SHARED((N, DH), jnp.float32),
            pltpu.VMEM((4, K), jnp.int32),
            pltpu.VMEM((4, K), jnp.int32),
            pltpu.VMEM((2, K, DH), jnp.float32),
            pltpu.VMEM((2, K, DH // 2), jnp.int32),
            pltpu.SemaphoreType.DMA,
            pltpu.SemaphoreType.DMA,
            pltpu.SemaphoreType.DMA,
            pltpu.SemaphoreType.DMA,
            pltpu.SemaphoreType.DMA,
            pltpu.SemaphoreType.DMA,
            pltpu.SemaphoreType.DMA,
            pltpu.SemaphoreType.DMA,
            pltpu.SemaphoreType.DMA,
        ],
    )
    def k(x2_hbm, eh_hbm, srcx_hbm, dst_hbm, out_hbm,
          acc, srcv, dstv, xbuf, ebuf, semx0, semx1, seme0, seme1,
          semsc, semi0, semi1, semi2, semi3):
        semx = [semx0, semx1]
        seme = [seme0, seme1]
        c = lax.axis_index("c")
        s = lax.axis_index("s")

        def zbody(r, carry):
            for g in range(DH // 16):
                xbuf[0, r, pl.ds(g * 16, 16)] = jnp.zeros((16,), jnp.float32)
            return carry
        lax.fori_loop(0, K, zbody, 0)
        row0 = s * RPT_A

        def _zero_rows(nrows):
            full = nrows // K
            for kk in range(full):
                pltpu.sync_copy(xbuf.at[0], acc.at[pl.ds(row0 + kk * K, K)])
            r = nrows - full * K
            if r:
                pltpu.sync_copy(xbuf.at[0, pl.ds(0, r)],
                                acc.at[pl.ds(row0 + full * K, r)])

        pl.when(s < NSUB - 1)(lambda: _zero_rows(RPT_A))
        pl.when(s == NSUB - 1)(lambda: _zero_rows(RPT_LAST))
        plsc.subcore_barrier()

        ebase0 = s * EPT
        semi = [semi0, semi1, semi2, semi3]
        row_hbm = c * NSUB + s

        def _idx_start(i, q):
            pltpu.async_copy(srcx_hbm.at[row_hbm, i], srcv.at[q], semi[q])
            pltpu.async_copy(dst_hbm.at[s, i], dstv.at[q], semi[q])

        def _idx_wait(q):
            pltpu.make_async_copy(srcx_hbm.at[0, 0], srcv.at[q], semi[q]).wait()
            pltpu.make_async_copy(dst_hbm.at[0, 0], dstv.at[q], semi[q]).wait()

        def _start(i, q, p):
            pltpu.async_copy(x2_hbm.at[srcv.at[q]], xbuf.at[p], semx[p])
            pltpu.async_copy(eh_hbm.at[pl.ds(c * E + ebase0 + i * K, K)],
                             ebuf.at[p], seme[p])

        _idx_start(0, 0)
        _idx_start(1, 1)
        _idx_wait(0)
        _start(0, 0, 0)

        def step(i, b, first=False):
            p = b % 2
            pn = 1 - p
            qn1 = (b + 1) % 4
            qn2 = (b + 2) % 4

            wait_sc = lambda: pltpu.make_async_copy(
                xbuf.at[pn], acc.at[dstv.at[pn]], semsc).wait()
            if first:
                pl.when(i > 0)(wait_sc)
            else:
                wait_sc()
            pl.when(i + 2 < NCH)(lambda: _idx_start(i + 2, qn2))

            def _pref():
                _idx_wait(qn1)
                _start(i + 1, qn1, pn)
            pl.when(i + 1 < NCH)(_pref)

            pltpu.make_async_copy(
                x2_hbm.at[srcv.at[b]], xbuf.at[p], semx[p]).wait()
            pltpu.make_async_copy(
                eh_hbm.at[pl.ds(0, K)], ebuf.at[p], seme[p]).wait()

            def rbody(r, cr):
                for g in range(DH // 32):
                    ev = ebuf[p, r, pl.ds(g * 16, 16)]
                    ee = lax.bitcast_convert_type(ev << 16, jnp.float32)
                    eo = lax.bitcast_convert_type(ev & (-65536), jnp.float32)
                    xe = xbuf[p, r, pl.ds(g * 32, 16)]
                    xo = xbuf[p, r, pl.ds(g * 32 + 16, 16)]
                    xbuf[p, r, pl.ds(g * 32, 16)] = jnp.maximum(xe + ee, 0.0)
                    xbuf[p, r, pl.ds(g * 32 + 16, 16)] = jnp.maximum(xo + eo, 0.0)
                return cr
            lax.fori_loop(0, K, rbody, 0)

            pltpu.async_copy(xbuf.at[p], acc.at[dstv.at[b]], semsc, add=True)

        def quad(j, carry):
            i0 = j * 4
            step(i0, 0, first=True)
            step(i0 + 1, 1)
            step(i0 + 2, 2)
            step(i0 + 3, 3)
            return carry
        lax.fori_loop(0, NCH // 4, quad, 0)
        step(NCH - 1, (NCH - 1) % 4)

        pltpu.make_async_copy(
            xbuf.at[(NCH - 1) % 2], acc.at[dstv.at[0]], semsc).wait()
        plsc.subcore_barrier()

        def _writeout(nrows):
            pltpu.sync_copy(acc.at[pl.ds(row0, nrows)],
                            out_hbm.at[pl.ds(c * N + row0, nrows)])

        pl.when(s < NSUB - 1)(lambda: _writeout(RPT_A))
        pl.when(s == NSUB - 1)(lambda: _writeout(RPT_LAST))

    return k(xbf, eh, srcx4, dst3)


def _mlp(x, h2, W1, b1r, W2, b2r):
    def body(x_ref, al_ref, ar_ref, w1_ref, b1_ref, w2_ref, b2_ref, o_ref):
        h = x_ref[...] + jnp.concatenate([al_ref[...], ar_ref[...]], axis=1)
        t = jnp.maximum(
            jnp.dot(h, w1_ref[...], preferred_element_type=jnp.float32)
            + b1_ref[...], 0.0)
        o_ref[...] = jnp.dot(t, w2_ref[...],
                             preferred_element_type=jnp.float32) + b2_ref[...]

    return pl.pallas_call(
        body,
        grid=(N // MB,),
        in_specs=[
            pl.BlockSpec((MB, D), lambda i: (i, 0)),
            pl.BlockSpec((MB, DH), lambda i: (i, 0)),
            pl.BlockSpec((MB, DH), lambda i: (N // MB + i, 0)),
            pl.BlockSpec((D, D), lambda i: (0, 0)),
            pl.BlockSpec((1, D), lambda i: (0, 0)),
            pl.BlockSpec((D, D), lambda i: (0, 0)),
            pl.BlockSpec((1, D), lambda i: (0, 0)),
        ],
        out_specs=pl.BlockSpec((MB, D), lambda i: (i, 0)),
        out_shape=jax.ShapeDtypeStruct((N, D), jnp.float32),
    )(x, h2, h2, W1, b1r, W2, b2r)


def kernel(x, edge_index, edge_attr, We, be, W1, b1, W2, b2):
    src = edge_index[0].astype(jnp.int32)
    dst = edge_index[1].astype(jnp.int32)
    src2 = src * 2
    srcx4 = jnp.concatenate([src2, src2 + 1]).reshape(2 * NSUB, NCH, K)
    dst3 = dst.reshape(NSUB, NCH, K)
    x2 = x.reshape(2 * N, DH)

    eh = _edge_linear(edge_attr, We[:, _PERM_INTL], be[_PERM_INTL].reshape(2, DH))
    eh = lax.bitcast_convert_type(
        eh.reshape(2 * E, DH // 2, 2), jnp.int32)
    h2 = _sc_aggregate(x2, eh, srcx4, dst3)
    return _mlp(x, h2, W1, b1.reshape(1, D), W2, b2.reshape(1, D))


# speedup vs baseline: 2.4669x; 2.4243x over previous
import functools

import jax
import jax.numpy as jnp
import numpy as np
from jax import lax
from jax.experimental import pallas as pl
from jax.experimental.pallas import tpu as pltpu
from jax.experimental.pallas import tpu_sc as plsc

N = 10000
E = 160000
D = 256
DH = 128
DE = 16

NSUB = 16
K = 80
EPT = E // NSUB
NCH = EPT // K
RPT_A = 632
RPT_LAST = N - 15 * RPT_A

EB = 2000
MB = 1000

def _edge_linear(edge_attr, We, be2):
    def body(a_ref, w_ref, b_ref, o_ref):
        v = jnp.dot(a_ref[...], w_ref[...],
                    preferred_element_type=jnp.float32) + b_ref[0]
        bits = lax.bitcast_convert_type(v, jnp.int32) + 32768
        words = [
            (bits[:, g * 32 + 16:g * 32 + 32] & (-65536))
            | lax.shift_right_logical(bits[:, g * 32:g * 32 + 16], 16)
            for g in range(DH // 32)
        ]
        o_ref[...] = jnp.concatenate(words, axis=1)

    return pl.pallas_call(
        body,
        grid=(2, E // EB),
        in_specs=[
            pl.BlockSpec((EB, DE), lambda c, i: (i, 0)),
            pl.BlockSpec((DE, DH), lambda c, i: (0, c)),
            pl.BlockSpec((1, 1, DH), lambda c, i: (c, 0, 0)),
        ],
        out_specs=pl.BlockSpec((EB, DH // 2), lambda c, i: (c * (E // EB) + i, 0)),
        out_shape=jax.ShapeDtypeStruct((2 * E, DH // 2), jnp.int32),
    )(edge_attr, We, be2.reshape(2, 1, DH))


def _sc_aggregate(xbf, eh, srcx4, dst3):
    mesh = plsc.VectorSubcoreMesh(core_axis_name="c", subcore_axis_name="s")

    @functools.partial(
        pl.kernel,
        out_type=jax.ShapeDtypeStruct((2 * N, DH), jnp.float32),
        mesh=mesh,
        scratch_types=[
            pltpu.VMEM_SHARED((N, DH), jnp.float32),
            pltpu.VMEM((4, K), jnp.int32),
            pltpu.VMEM((4, K), jnp.int32),
            pltpu.VMEM((2, K, DH), jnp.float32),
            pltpu.VMEM((2, K, DH // 2), jnp.int32),
            pltpu.SemaphoreType.DMA,
            pltpu.SemaphoreType.DMA,
            pltpu.SemaphoreType.DMA,
            pltpu.SemaphoreType.DMA,
            pltpu.SemaphoreType.DMA,
            pltpu.SemaphoreType.DMA,
            pltpu.SemaphoreType.DMA,
            pltpu.SemaphoreType.DMA,
            pltpu.SemaphoreType.DMA,
        ],
    )
    def k(x2_hbm, eh_hbm, srcx_hbm, dst_hbm, out_hbm,
          acc, srcv, dstv, xbuf, ebuf, semx0, semx1, seme0, seme1,
          semsc, semi0, semi1, semi2, semi3):
        semx = [semx0, semx1]
        seme = [seme0, seme1]
        c = lax.axis_index("c")
        s = lax.axis_index("s")

        def zbody(r, carry):
            for g in range(DH // 16):
                xbuf[0, r, pl.ds(g * 16, 16)] = jnp.zeros((16,), jnp.float32)
            return carry
        lax.fori_loop(0, K, zbody, 0)
        row0 = s * RPT_A

        def _zero_rows(nrows):
            full = nrows // K
            for kk in range(full):
                pltpu.sync_copy(xbuf.at[0], acc.at[pl.ds(row0 + kk * K, K)])
            r = nrows - full * K
            if r:
                pltpu.sync_copy(xbuf.at[0, pl.ds(0, r)],
                                acc.at[pl.ds(row0 + full * K, r)])

        pl.when(s < NSUB - 1)(lambda: _zero_rows(RPT_A))
        pl.when(s == NSUB - 1)(lambda: _zero_rows(RPT_LAST))
        plsc.subcore_barrier()

        ebase0 = s * EPT
        semi = [semi0, semi1, semi2, semi3]
        row_hbm = c * NSUB + s

        def _idx_start(i, q):
            pltpu.async_copy(srcx_hbm.at[row_hbm, i], srcv.at[q], semi[q])
            pltpu.async_copy(dst_hbm.at[s, i], dstv.at[q], semi[q])

        def _idx_wait(q):
            pltpu.make_async_copy(srcx_hbm.at[0, 0], srcv.at[q], semi[q]).wait()
            pltpu.make_async_copy(dst_hbm.at[0, 0], dstv.at[q], semi[q]).wait()

        def _start(i, q, p):
            pltpu.async_copy(x2_hbm.at[srcv.at[q]], xbuf.at[p], semx[p])
            pltpu.async_copy(eh_hbm.at[pl.ds(c * E + ebase0 + i * K, K)],
                             ebuf.at[p], seme[p])

        _idx_start(0, 0)
        _idx_start(1, 1)
        _idx_wait(0)
        _start(0, 0, 0)

        def step(i, b, first=False):
            p = b % 2
            pn = 1 - p
            qn1 = (b + 1) % 4
            qn2 = (b + 2) % 4

            wait_sc = lambda: pltpu.make_async_copy(
                xbuf.at[pn], acc.at[dstv.at[pn]], semsc).wait()
            if first:
                pl.when(i > 0)(wait_sc)
            else:
                wait_sc()
            pl.when(i + 2 < NCH)(lambda: _idx_start(i + 2, qn2))

            def _pref():
                _idx_wait(qn1)
                _start(i + 1, qn1, pn)
            pl.when(i + 1 < NCH)(_pref)

            pltpu.make_async_copy(
                x2_hbm.at[srcv.at[b]], xbuf.at[p], semx[p]).wait()
            pltpu.make_async_copy(
                eh_hbm.at[pl.ds(0, K)], ebuf.at[p], seme[p]).wait()

            def rbody(r, cr):
                for g in range(DH // 32):
                    ev = ebuf[p, r, pl.ds(g * 16, 16)]
                    ee = lax.bitcast_convert_type(ev << 16, jnp.float32)
                    eo = lax.bitcast_convert_type(ev & (-65536), jnp.float32)
                    xe = xbuf[p, r, pl.ds(g * 32, 16)]
                    xo = xbuf[p, r, pl.ds(g * 32 + 16, 16)]
                    xbuf[p, r, pl.ds(g * 32, 16)] = jnp.maximum(xe + ee, 0.0)
                    xbuf[p, r, pl.ds(g * 32 + 16, 16)] = jnp.maximum(xo + eo, 0.0)
                return cr
            lax.fori_loop(0, K, rbody, 0)

            pltpu.async_copy(xbuf.at[p], acc.at[dstv.at[b]], semsc, add=True)

        def quad(j, carry):
            i0 = j * 4
            step(i0, 0, first=True)
            step(i0 + 1, 1)
            step(i0 + 2, 2)
            step(i0 + 3, 3)
            return carry
        lax.fori_loop(0, NCH // 4, quad, 0)
        step(NCH - 1, (NCH - 1) % 4)

        pltpu.make_async_copy(
            xbuf.at[(NCH - 1) % 2], acc.at[dstv.at[0]], semsc).wait()
        plsc.subcore_barrier()

        def _writeout(nrows):
            pltpu.sync_copy(acc.at[pl.ds(row0, nrows)],
                            out_hbm.at[pl.ds(c * N + row0, nrows)])

        pl.when(s < NSUB - 1)(lambda: _writeout(RPT_A))
        pl.when(s == NSUB - 1)(lambda: _writeout(RPT_LAST))

    return k(xbf, eh, srcx4, dst3)


def _mlp(x, h2, W1, b1r, W2, b2r):
    def body(x_ref, al_ref, ar_ref, w1_ref, b1_ref, w2_ref, b2_ref, o_ref):
        h = x_ref[...] + jnp.concatenate([al_ref[...], ar_ref[...]], axis=1)
        t = jnp.maximum(
            jnp.dot(h, w1_ref[...], preferred_element_type=jnp.float32)
            + b1_ref[...], 0.0)
        o_ref[...] = jnp.dot(t, w2_ref[...],
                             preferred_element_type=jnp.float32) + b2_ref[...]

    return pl.pallas_call(
        body,
        grid=(N // MB,),
        in_specs=[
            pl.BlockSpec((MB, D), lambda i: (i, 0)),
            pl.BlockSpec((MB, DH), lambda i: (i, 0)),
            pl.BlockSpec((MB, DH), lambda i: (N // MB + i, 0)),
            pl.BlockSpec((D, D), lambda i: (0, 0)),
            pl.BlockSpec((1, D), lambda i: (0, 0)),
            pl.BlockSpec((D, D), lambda i: (0, 0)),
            pl.BlockSpec((1, D), lambda i: (0, 0)),
        ],
        out_specs=pl.BlockSpec((MB, D), lambda i: (i, 0)),
        out_shape=jax.ShapeDtypeStruct((N, D), jnp.float32),
    )(x, h2, h2, W1, b1r, W2, b2r)


def kernel(x, edge_index, edge_attr, We, be, W1, b1, W2, b2):
    src = edge_index[0].astype(jnp.int32)
    dst = edge_index[1].astype(jnp.int32)
    src2 = src * 2
    srcx4 = jnp.concatenate([src2, src2 + 1]).reshape(2 * NSUB, NCH, K)
    dst3 = dst.reshape(NSUB, NCH, K)
    x2 = x.reshape(2 * N, DH)

    eh = _edge_linear(edge_attr, We, be.reshape(2, DH))
    h2 = _sc_aggregate(x2, eh, srcx4, dst3)
    return _mlp(x, h2, W1, b1.reshape(1, D), W2, b2.reshape(1, D))


# speedup vs baseline: 3.2530x; 1.3186x over previous
import functools

import jax
import jax.numpy as jnp
import numpy as np
from jax import lax
from jax.experimental import pallas as pl
from jax.experimental.pallas import tpu as pltpu
from jax.experimental.pallas import tpu_sc as plsc

N = 10000
E = 160000
D = 256
DH = 128
DE = 16

NSUB = 16
K = 80
EPT = E // NSUB
NCH = EPT // K
RPT_A = 632
RPT_LAST = N - 15 * RPT_A

EB = 2000
MB = 1000

def _edge_linear(edge_attr, We, be2):
    def body(a_ref, w_ref, b_ref, o_ref):
        v = jnp.dot(a_ref[...], w_ref[...],
                    preferred_element_type=jnp.float32) + b_ref[0]
        bits = lax.bitcast_convert_type(v, jnp.int32) + 32768
        for g in range(EB // K):
            lo = lax.shift_right_logical(
                bits[g * K:g * K + K // 2, :], 16)
            hi = bits[g * K + K // 2:(g + 1) * K, :] & (-65536)
            o_ref[pl.ds(g * (K // 2), K // 2), :] = hi | lo

    return pl.pallas_call(
        body,
        grid=(2, E // EB),
        in_specs=[
            pl.BlockSpec((EB, DE), lambda c, i: (i, 0)),
            pl.BlockSpec((DE, DH), lambda c, i: (0, c)),
            pl.BlockSpec((1, 1, DH), lambda c, i: (c, 0, 0)),
        ],
        out_specs=pl.BlockSpec((EB // 2, DH), lambda c, i: (c * (E // EB) + i, 0)),
        out_shape=jax.ShapeDtypeStruct((E, DH), jnp.int32),
    )(edge_attr, We, be2.reshape(2, 1, DH))


def _sc_aggregate(xbf, eh, srcx4, dst3):
    mesh = plsc.VectorSubcoreMesh(core_axis_name="c", subcore_axis_name="s")

    @functools.partial(
        pl.kernel,
        out_type=jax.ShapeDtypeStruct((2 * N, DH), jnp.float32),
        mesh=mesh,
        scratch_types=[
            pltpu.VMEM_SHARED((N, DH), jnp.float32),
            pltpu.VMEM((4, K), jnp.int32),
            pltpu.VMEM((4, K), jnp.int32),
            pltpu.VMEM((2, K, DH), jnp.float32),
            pltpu.VMEM((2, K // 2, DH), jnp.int32),
            pltpu.SemaphoreType.DMA,
            pltpu.SemaphoreType.DMA,
            pltpu.SemaphoreType.DMA,
            pltpu.SemaphoreType.DMA,
            pltpu.SemaphoreType.DMA,
            pltpu.SemaphoreType.DMA,
            pltpu.SemaphoreType.DMA,
            pltpu.SemaphoreType.DMA,
            pltpu.SemaphoreType.DMA,
        ],
    )
    def k(x2_hbm, eh_hbm, srcx_hbm, dst_hbm, out_hbm,
          acc, srcv, dstv, xbuf, ebuf, semx0, semx1, seme0, seme1,
          semsc, semi0, semi1, semi2, semi3):
        semx = [semx0, semx1]
        seme = [seme0, seme1]
        c = lax.axis_index("c")
        s = lax.axis_index("s")

        def zbody(r, carry):
            for g in range(DH // 16):
                xbuf[0, r, pl.ds(g * 16, 16)] = jnp.zeros((16,), jnp.float32)
            return carry
        lax.fori_loop(0, K, zbody, 0)
        row0 = s * RPT_A

        def _zero_rows(nrows):
            full = nrows // K
            for kk in range(full):
                pltpu.sync_copy(xbuf.at[0], acc.at[pl.ds(row0 + kk * K, K)])
            r = nrows - full * K
            if r:
                pltpu.sync_copy(xbuf.at[0, pl.ds(0, r)],
                                acc.at[pl.ds(row0 + full * K, r)])

        pl.when(s < NSUB - 1)(lambda: _zero_rows(RPT_A))
        pl.when(s == NSUB - 1)(lambda: _zero_rows(RPT_LAST))
        plsc.subcore_barrier()

        ebase0 = s * EPT
        semi = [semi0, semi1, semi2, semi3]
        row_hbm = c * NSUB + s

        def _idx_start(i, q):
            pltpu.async_copy(srcx_hbm.at[row_hbm, i], srcv.at[q], semi[q])
            pltpu.async_copy(dst_hbm.at[s, i], dstv.at[q], semi[q])

        def _idx_wait(q):
            pltpu.make_async_copy(srcx_hbm.at[0, 0], srcv.at[q], semi[q]).wait()
            pltpu.make_async_copy(dst_hbm.at[0, 0], dstv.at[q], semi[q]).wait()

        wbase0 = c * (E // 2) + s * (EPT // 2)

        def _start(i, q, p):
            pltpu.async_copy(x2_hbm.at[srcv.at[q]], xbuf.at[p], semx[p])
            pltpu.async_copy(eh_hbm.at[pl.ds(wbase0 + i * (K // 2), K // 2)],
                             ebuf.at[p], seme[p])

        _idx_start(0, 0)
        _idx_start(1, 1)
        _idx_wait(0)
        _start(0, 0, 0)

        def step(i, b, first=False):
            p = b % 2
            pn = 1 - p
            qn1 = (b + 1) % 4
            qn2 = (b + 2) % 4

            wait_sc = lambda: pltpu.make_async_copy(
                xbuf.at[pn], acc.at[dstv.at[pn]], semsc).wait()
            if first:
                pl.when(i > 0)(wait_sc)
            else:
                wait_sc()
            pl.when(i + 2 < NCH)(lambda: _idx_start(i + 2, qn2))

            def _pref():
                _idx_wait(qn1)
                _start(i + 1, qn1, pn)
            pl.when(i + 1 < NCH)(_pref)

            pltpu.make_async_copy(
                x2_hbm.at[srcv.at[b]], xbuf.at[p], semx[p]).wait()
            pltpu.make_async_copy(
                eh_hbm.at[pl.ds(0, K // 2)], ebuf.at[p], seme[p]).wait()

            def rbody(t, cr):
                t2 = t + K // 2
                for g in range(DH // 16):
                    sl = pl.ds(g * 16, 16)
                    ev = ebuf[p, t, sl]
                    ee = lax.bitcast_convert_type(ev << 16, jnp.float32)
                    eo = lax.bitcast_convert_type(ev & (-65536), jnp.float32)
                    xbuf[p, t, sl] = jnp.maximum(xbuf[p, t, sl] + ee, 0.0)
                    xbuf[p, t2, sl] = jnp.maximum(xbuf[p, t2, sl] + eo, 0.0)
                return cr
            lax.fori_loop(0, K // 2, rbody, 0)

            pltpu.async_copy(xbuf.at[p], acc.at[dstv.at[b]], semsc, add=True)

        def quad(j, carry):
            i0 = j * 4
            step(i0, 0, first=True)
            step(i0 + 1, 1)
            step(i0 + 2, 2)
            step(i0 + 3, 3)
            return carry
        lax.fori_loop(0, NCH // 4, quad, 0)
        step(NCH - 1, (NCH - 1) % 4)

        pltpu.make_async_copy(
            xbuf.at[(NCH - 1) % 2], acc.at[dstv.at[0]], semsc).wait()
        plsc.subcore_barrier()

        def _writeout(nrows):
            pltpu.sync_copy(acc.at[pl.ds(row0, nrows)],
                            out_hbm.at[pl.ds(c * N + row0, nrows)])

        pl.when(s < NSUB - 1)(lambda: _writeout(RPT_A))
        pl.when(s == NSUB - 1)(lambda: _writeout(RPT_LAST))

    return k(xbf, eh, srcx4, dst3)


def _mlp(x, h2, W1, b1r, W2, b2r):
    def body(x_ref, al_ref, ar_ref, w1_ref, b1_ref, w2_ref, b2_ref, o_ref):
        h = x_ref[...] + jnp.concatenate([al_ref[...], ar_ref[...]], axis=1)
        t = jnp.maximum(
            jnp.dot(h, w1_ref[...], preferred_element_type=jnp.float32)
            + b1_ref[...], 0.0)
        o_ref[...] = jnp.dot(t, w2_ref[...],
                             preferred_element_type=jnp.float32) + b2_ref[...]

    return pl.pallas_call(
        body,
        grid=(N // MB,),
        in_specs=[
            pl.BlockSpec((MB, D), lambda i: (i, 0)),
            pl.BlockSpec((MB, DH), lambda i: (i, 0)),
            pl.BlockSpec((MB, DH), lambda i: (N // MB + i, 0)),
            pl.BlockSpec((D, D), lambda i: (0, 0)),
            pl.BlockSpec((1, D), lambda i: (0, 0)),
            pl.BlockSpec((D, D), lambda i: (0, 0)),
            pl.BlockSpec((1, D), lambda i: (0, 0)),
        ],
        out_specs=pl.BlockSpec((MB, D), lambda i: (i, 0)),
        out_shape=jax.ShapeDtypeStruct((N, D), jnp.float32),
    )(x, h2, h2, W1, b1r, W2, b2r)


def kernel(x, edge_index, edge_attr, We, be, W1, b1, W2, b2):
    src = edge_index[0].astype(jnp.int32)
    dst = edge_index[1].astype(jnp.int32)
    src2 = src * 2
    srcx4 = jnp.concatenate([src2, src2 + 1]).reshape(2 * NSUB, NCH, K)
    dst3 = dst.reshape(NSUB, NCH, K)
    x2 = x.reshape(2 * N, DH)

    eh = _edge_linear(edge_attr, We, be.reshape(2, DH))
    h2 = _sc_aggregate(x2, eh, srcx4, dst3)
    return _mlp(x, h2, W1, b1.reshape(1, D), W2, b2.reshape(1, D))


# speedup vs baseline: 3.3474x; 1.0290x over previous
import functools

import jax
import jax.numpy as jnp
import numpy as np
from jax import lax
from jax.experimental import pallas as pl
from jax.experimental.pallas import tpu as pltpu
from jax.experimental.pallas import tpu_sc as plsc

N = 10000
E = 160000
D = 256
DH = 128
DE = 16

NSUB = 16
K = 80
EPT = E // NSUB
NCH = EPT // K
RPT_A = 632
RPT_LAST = N - 15 * RPT_A

EB = 2000
MB = 2000

def _edge_linear(edge_attr, We, be2):
    def body(a_ref, w_ref, b_ref, o_ref):
        v = jnp.dot(a_ref[...], w_ref[...],
                    preferred_element_type=jnp.float32) + b_ref[0]
        bits = lax.bitcast_convert_type(v, jnp.int32) + 32768
        for g in range(EB // K):
            lo = lax.shift_right_logical(
                bits[g * K:g * K + K // 2, :], 16)
            hi = bits[g * K + K // 2:(g + 1) * K, :] & (-65536)
            o_ref[pl.ds(g * (K // 2), K // 2), :] = hi | lo

    return pl.pallas_call(
        body,
        grid=(2, E // EB),
        in_specs=[
            pl.BlockSpec((EB, DE), lambda c, i: (i, 0)),
            pl.BlockSpec((DE, DH), lambda c, i: (0, c)),
            pl.BlockSpec((1, 1, DH), lambda c, i: (c, 0, 0)),
        ],
        out_specs=pl.BlockSpec((EB // 2, DH), lambda c, i: (c * (E // EB) + i, 0)),
        out_shape=jax.ShapeDtypeStruct((E, DH), jnp.int32),
    )(edge_attr, We, be2.reshape(2, 1, DH))


def _sc_aggregate(xbf, eh, ei4):
    mesh = plsc.VectorSubcoreMesh(core_axis_name="c", subcore_axis_name="s")

    @functools.partial(
        pl.kernel,
        out_type=jax.ShapeDtypeStruct((2 * N, DH), jnp.float32),
        mesh=mesh,
        scratch_types=[
            pltpu.VMEM_SHARED((N, DH), jnp.float32),
            pltpu.VMEM((4, K), jnp.int32),
            pltpu.VMEM((4, K), jnp.int32),
            pltpu.VMEM((2, K, DH), jnp.float32),
            pltpu.VMEM((2, K // 2, DH), jnp.int32),
            pltpu.SemaphoreType.DMA,
            pltpu.SemaphoreType.DMA,
            pltpu.SemaphoreType.DMA,
            pltpu.SemaphoreType.DMA,
            pltpu.SemaphoreType.DMA,
            pltpu.SemaphoreType.DMA,
            pltpu.SemaphoreType.DMA,
            pltpu.SemaphoreType.DMA,
            pltpu.SemaphoreType.DMA,
        ],
    )
    def k(x2_hbm, eh_hbm, ei_hbm, out_hbm,
          acc, srcv, dstv, xbuf, ebuf, semx0, semx1, seme0, seme1,
          semsc, semi0, semi1, semi2, semi3):
        semx = [semx0, semx1]
        seme = [seme0, seme1]
        c = lax.axis_index("c")
        s = lax.axis_index("s")

        def zbody(r, carry):
            for g in range(DH // 16):
                xbuf[0, r, pl.ds(g * 16, 16)] = jnp.zeros((16,), jnp.float32)
            return carry
        lax.fori_loop(0, K, zbody, 0)
        row0 = s * RPT_A

        def _zero_rows(nrows):
            full = nrows // K
            for kk in range(full):
                pltpu.sync_copy(xbuf.at[0], acc.at[pl.ds(row0 + kk * K, K)])
            r = nrows - full * K
            if r:
                pltpu.sync_copy(xbuf.at[0, pl.ds(0, r)],
                                acc.at[pl.ds(row0 + full * K, r)])

        pl.when(s < NSUB - 1)(lambda: _zero_rows(RPT_A))
        pl.when(s == NSUB - 1)(lambda: _zero_rows(RPT_LAST))
        plsc.subcore_barrier()

        ebase0 = s * EPT
        semi = [semi0, semi1, semi2, semi3]

        def _idx_start(i, q):
            pltpu.async_copy(ei_hbm.at[0, s, i], srcv.at[q], semi[q])
            pltpu.async_copy(ei_hbm.at[1, s, i], dstv.at[q], semi[q])

        def _idx_wait(q):
            pltpu.make_async_copy(ei_hbm.at[0, 0, 0], srcv.at[q], semi[q]).wait()
            pltpu.make_async_copy(ei_hbm.at[0, 0, 0], dstv.at[q], semi[q]).wait()
            for g in range(K // 16):
                sl = pl.ds(g * 16, 16)
                srcv[q, sl] = srcv[q, sl] * 2 + c

        wbase0 = c * (E // 2) + s * (EPT // 2)

        def _start(i, q, p):
            pltpu.async_copy(x2_hbm.at[srcv.at[q]], xbuf.at[p], semx[p])
            pltpu.async_copy(eh_hbm.at[pl.ds(wbase0 + i * (K // 2), K // 2)],
                             ebuf.at[p], seme[p])

        _idx_start(0, 0)
        _idx_start(1, 1)
        _idx_wait(0)
        _start(0, 0, 0)

        def step(i, b, first=False):
            p = b % 2
            pn = 1 - p
            qn1 = (b + 1) % 4
            qn2 = (b + 2) % 4

            wait_sc = lambda: pltpu.make_async_copy(
                xbuf.at[pn], acc.at[dstv.at[pn]], semsc).wait()
            if first:
                pl.when(i > 0)(wait_sc)
            else:
                wait_sc()
            pl.when(i + 2 < NCH)(lambda: _idx_start(i + 2, qn2))

            def _pref():
                _idx_wait(qn1)
                _start(i + 1, qn1, pn)
            pl.when(i + 1 < NCH)(_pref)

            pltpu.make_async_copy(
                x2_hbm.at[srcv.at[b]], xbuf.at[p], semx[p]).wait()
            pltpu.make_async_copy(
                eh_hbm.at[pl.ds(0, K // 2)], ebuf.at[p], seme[p]).wait()

            def rbody(t, cr):
                t2 = t + K // 2
                for g in range(DH // 16):
                    sl = pl.ds(g * 16, 16)
                    ev = ebuf[p, t, sl]
                    ee = lax.bitcast_convert_type(ev << 16, jnp.float32)
                    eo = lax.bitcast_convert_type(ev & (-65536), jnp.float32)
                    xbuf[p, t, sl] = jnp.maximum(xbuf[p, t, sl] + ee, 0.0)
                    xbuf[p, t2, sl] = jnp.maximum(xbuf[p, t2, sl] + eo, 0.0)
                return cr
            lax.fori_loop(0, K // 2, rbody, 0)

            pltpu.async_copy(xbuf.at[p], acc.at[dstv.at[b]], semsc, add=True)

        def quad(j, carry):
            i0 = j * 4
            step(i0, 0, first=True)
            step(i0 + 1, 1)
            step(i0 + 2, 2)
            step(i0 + 3, 3)
            return carry
        lax.fori_loop(0, NCH // 4, quad, 0)
        step(NCH - 1, (NCH - 1) % 4)

        pltpu.make_async_copy(
            xbuf.at[(NCH - 1) % 2], acc.at[dstv.at[0]], semsc).wait()
        plsc.subcore_barrier()

        def _writeout(nrows):
            pltpu.sync_copy(acc.at[pl.ds(row0, nrows)],
                            out_hbm.at[pl.ds(c * N + row0, nrows)])

        pl.when(s < NSUB - 1)(lambda: _writeout(RPT_A))
        pl.when(s == NSUB - 1)(lambda: _writeout(RPT_LAST))

    return k(xbf, eh, ei4)


def _mlp(x, h2, W1, b1r, W2, b2r):
    def body(x_ref, a_ref, w1_ref, b1_ref, w2_ref, b2_ref, o_ref):
        h = x_ref[...] + jnp.concatenate([a_ref[0], a_ref[1]], axis=1)
        t = jnp.maximum(
            jnp.dot(h, w1_ref[...], preferred_element_type=jnp.float32)
            + b1_ref[...], 0.0)
        o_ref[...] = jnp.dot(t, w2_ref[...],
                             preferred_element_type=jnp.float32) + b2_ref[...]

    return pl.pallas_call(
        body,
        grid=(N // MB,),
        in_specs=[
            pl.BlockSpec((MB, D), lambda i: (i, 0)),
            pl.BlockSpec((2, MB, DH), lambda i: (0, i, 0)),
            pl.BlockSpec((D, D), lambda i: (0, 0)),
            pl.BlockSpec((1, D), lambda i: (0, 0)),
            pl.BlockSpec((D, D), lambda i: (0, 0)),
            pl.BlockSpec((1, D), lambda i: (0, 0)),
        ],
        out_specs=pl.BlockSpec((MB, D), lambda i: (i, 0)),
        out_shape=jax.ShapeDtypeStruct((N, D), jnp.float32),
    )(x, h2, W1, b1r, W2, b2r)


def kernel(x, edge_index, edge_attr, We, be, W1, b1, W2, b2):
    ei4 = edge_index.astype(jnp.int32).reshape(2, NSUB, NCH, K)
    x2 = x.reshape(2 * N, DH)

    eh = _edge_linear(edge_attr, We, be.reshape(2, DH))
    h2 = _sc_aggregate(x2, eh, ei4)
    return _mlp(x, h2.reshape(2, N, DH), W1, b1.reshape(1, D),
                W2, b2.reshape(1, D))
